# Initial kernel scaffold; baseline (speedup 1.0000x reference)
import jax
import jax.numpy as jnp
from jax.experimental import pallas as pl


def kernel(query_tokens, rag_scores, attention_mask):
    raise NotImplementedError("write your pallas kernel here")


# speedup vs baseline: 1.8953x; 1.8953x over previous
import functools

import jax
import jax.numpy as jnp
from jax import lax
from jax.experimental import pallas as pl
from jax.experimental.pallas import tpu as pltpu
from jax.experimental.pallas import tpu_sc as plsc

B = 128
S = 8192
L = 16
NBINS = 256


def _body(tok_hbm, sc_hbm, mask_hbm, out_hbm,
          tok_v, sc_v, mask_v, key_a, key_b, val_a, val_b,
          hist, ctr, sums_s, g_s, num_workers):
    lane = lax.broadcasted_iota(jnp.int32, (L,), 0)
    wid = lax.axis_index("s") * 2 + lax.axis_index("c")
    rows_per_w = B // num_workers

    def _zero_hist(d, c):
        hist[d, :] = jnp.zeros((L,), jnp.int32)
        return c
    lax.fori_loop(0, NBINS, _zero_hist, 0)

    def do_row(r, carry_row):
        row = wid * rows_per_w + r
        pltpu.sync_copy(tok_hbm.at[row], tok_v)
        pltpu.sync_copy(sc_hbm.at[row], sc_v)
        pltpu.sync_copy(mask_hbm.at[row], mask_v)

        def scan_body(v, carry):
            off_k, off_t = carry
            base = S - L * (v + 1)
            scv = jnp.flip(sc_v[pl.ds(base, L)], axis=0)
            tkv = jnp.flip(tok_v[pl.ds(base, L)], axis=0)
            mkv = jnp.flip(mask_v[pl.ds(base, L)], axis=0)
            keep = mkv != 0
            bits = lax.bitcast_convert_type(scv, jnp.int32)
            pos_key = jnp.bitwise_and(jnp.bitwise_not(bits),
                                      jnp.int32(0x7FFFFFFF))
            key = jnp.where(bits < 0, bits, pos_key)
            ki = plsc.cumsum(keep.astype(jnp.int32))
            pos_k = off_k + ki - 1
            plsc.store_scatter(key_a, [pos_k], key, mask=keep)
            plsc.store_scatter(val_a, [pos_k], tkv, mask=keep)
            nk = ki[L - 1]
            drop = jnp.logical_not(keep)
            ti = plsc.cumsum(drop.astype(jnp.int32))
            pos_t = off_t + ti - 1
            plsc.store_scatter(val_b, [jnp.int32(S + L - 1) - pos_t], tkv,
                               mask=drop)
            return (off_k + nk, off_t + (L - nk))

        m_cnt, d_cnt = lax.fori_loop(0, S // L, scan_body,
                                     (jnp.int32(0), jnp.int32(0)))

        pad = (-m_cnt) & (L - 1)
        plsc.store_scatter(key_a, [m_cnt + lane],
                           jnp.full((L,), -1, jnp.int32), mask=lane < pad)
        mp = m_cnt + pad
        chunk = mp // L

        def do_pass(kb_s, vb_s, kb_d, vb_d, shift):
            lane_c = lane * chunk

            def hist_body(v, c):
                idx = lane_c + v
                k = plsc.load_gather(kb_s, [idx])
                d = jnp.bitwise_and(lax.shift_right_logical(k, shift),
                                    jnp.int32(NBINS - 1))
                plsc.addupdate_scatter(hist, [d, lane],
                                       jnp.ones((L,), jnp.int32))
                return c
            lax.fori_loop(0, chunk, hist_body, 0)

            def sum_body(d, c):
                sums_s[d] = jnp.sum(hist[d, :])
                return c
            lax.fori_loop(0, NBINS, sum_body, 0)

            def g_body(d, g):
                g_s[d] = g
                return g + sums_s[d]
            lax.fori_loop(0, NBINS, g_body, jnp.int32(0))

            def ctr_body(d, c):
                h = hist[d, :]
                cs = plsc.cumsum(h)
                ctr[d, :] = cs - h + jnp.full((L,), g_s[d], jnp.int32)
                hist[d, :] = jnp.zeros((L,), jnp.int32)
                return c
            lax.fori_loop(0, NBINS, ctr_body, 0)

            def scat_body(v, c):
                idx = lane_c + v
                k = plsc.load_gather(kb_s, [idx])
                val = plsc.load_gather(vb_s, [idx])
                d = jnp.bitwise_and(lax.shift_right_logical(k, shift),
                                    jnp.int32(NBINS - 1))
                pos = plsc.load_gather(ctr, [d, lane])
                plsc.store_scatter(kb_d, [pos], k)
                plsc.store_scatter(vb_d, [pos], val)
                plsc.store_scatter(ctr, [d, lane], pos + 1)
                return c
            lax.fori_loop(0, chunk, scat_body, 0)

        do_pass(key_a, val_a, key_b, val_b, 0)
        do_pass(key_b, val_b, key_a, val_a, 8)
        do_pass(key_a, val_a, key_b, val_b, 16)
        do_pass(key_b, val_b, key_a, val_a, 24)

        n_tail = (d_cnt + L - 1) // L

        def tail_body(j, c):
            t = jnp.flip(val_b[pl.ds(S + L - L * (j + 1), L)], axis=0)
            plsc.store_scatter(val_a, [m_cnt + j * L + lane], t)
            return c
        lax.fori_loop(0, n_tail, tail_body, 0)

        pltpu.sync_copy(val_a.at[pl.ds(0, S)], out_hbm.at[row])
        return carry_row

    lax.fori_loop(0, rows_per_w, do_row, 0)


@jax.jit
def kernel(query_tokens, rag_scores, attention_mask):
    info = plsc.get_sparse_core_info()
    num_workers = info.num_cores * info.num_subcores
    mesh = plsc.VectorSubcoreMesh(core_axis_name="c", subcore_axis_name="s")
    body = functools.partial(_body, num_workers=num_workers)
    fn = pl.kernel(
        body,
        out_type=jax.ShapeDtypeStruct((B, S), jnp.int32),
        mesh=mesh,
        compiler_params=pltpu.CompilerParams(needs_layout_passes=False),
        scratch_types=[
            pltpu.VMEM((S,), jnp.int32),
            pltpu.VMEM((S,), jnp.float32),
            pltpu.VMEM((S,), jnp.int32),
            pltpu.VMEM((S + L,), jnp.int32),
            pltpu.VMEM((S + L,), jnp.int32),
            pltpu.VMEM((S + L,), jnp.int32),
            pltpu.VMEM((S + L,), jnp.int32),
            pltpu.VMEM((NBINS, L), jnp.int32),
            pltpu.VMEM((NBINS, L), jnp.int32),
            pltpu.SMEM((NBINS,), jnp.int32),
            pltpu.SMEM((NBINS,), jnp.int32),
        ],
    )
    return fn(query_tokens, rag_scores, attention_mask)


# speedup vs baseline: 2.4514x; 1.2934x over previous
import functools

import jax
import jax.numpy as jnp
from jax import lax
from jax.experimental import pallas as pl
from jax.experimental.pallas import tpu as pltpu
from jax.experimental.pallas import tpu_sc as plsc

B = 128
S = 8192
L = 16
NBINS = 256
PAD = 64


def _body(tok_hbm, sc_hbm, mask_hbm, out_hbm,
          tok_v, sc_v, mask_v, key_a, key_b, val_a, val_b,
          hist, ctr, sums_s, g_s, num_workers):
    lane = lax.broadcasted_iota(jnp.int32, (L,), 0)
    wid = lax.axis_index("s") * 2 + lax.axis_index("c")
    rows_per_w = B // num_workers

    @plsc.parallel_loop(0, NBINS, unroll=8)
    def _zero_hist(d):
        hist[d, :] = jnp.zeros((L,), jnp.int32)

    def do_row(r, carry_row):
        row = wid * rows_per_w + r
        pltpu.sync_copy(tok_hbm.at[row], tok_v)
        pltpu.sync_copy(sc_hbm.at[row], sc_v)
        pltpu.sync_copy(mask_hbm.at[row], mask_v)

        @plsc.parallel_loop(0, S // L, unroll=4,
                            carry=(jnp.int32(0), jnp.int32(0)))
        def scan_counts(v, carry):
            off_k, off_t = carry
            base = S - L * (v + 1)
            scv = jnp.flip(sc_v[pl.ds(base, L)], axis=0)
            tkv = jnp.flip(tok_v[pl.ds(base, L)], axis=0)
            mkv = jnp.flip(mask_v[pl.ds(base, L)], axis=0)
            keep = mkv != 0
            bits = lax.bitcast_convert_type(scv, jnp.int32)
            pos_key = jnp.bitwise_and(jnp.bitwise_not(bits),
                                      jnp.int32(0x7FFFFFFF))
            key = jnp.where(bits < 0, bits, pos_key)
            ki = plsc.cumsum(keep.astype(jnp.int32))
            pos_k = off_k + ki - 1
            plsc.store_scatter(key_a, [pos_k], key, mask=keep)
            plsc.store_scatter(val_a, [pos_k], tkv, mask=keep)
            nk = ki[L - 1]
            drop = jnp.logical_not(keep)
            pos_t = off_t + lane - ki
            plsc.store_scatter(val_b, [jnp.int32(S + PAD - 1) - pos_t], tkv,
                               mask=drop)
            return (off_k + nk, off_t + (L - nk))

        m_cnt, d_cnt = scan_counts

        pad = (-m_cnt) & (PAD - 1)
        for u in range(PAD // L):
            plsc.store_scatter(key_a, [m_cnt + u * L + lane],
                               jnp.full((L,), -1, jnp.int32),
                               mask=(u * L + lane) < pad)
        mp = m_cnt + pad
        chunk = mp // L

        def do_pass(kb_s, vb_s, kb_d, vb_d, shift):
            lane_c = lane * chunk

            @plsc.parallel_loop(0, chunk, unroll=4)
            def hist_loop(v):
                idx = lane_c + v
                k = plsc.load_gather(kb_s, [idx])
                d = jnp.bitwise_and(lax.shift_right_logical(k, shift),
                                    jnp.int32(NBINS - 1))
                plsc.addupdate_scatter(hist, [d, lane],
                                       jnp.ones((L,), jnp.int32))

            @plsc.parallel_loop(0, NBINS, unroll=8)
            def presum_loop(d):
                h = hist[d, :]
                cs = plsc.cumsum(h)
                ctr[d, :] = cs - h
                sums_s[d] = cs[L - 1]
                hist[d, :] = jnp.zeros((L,), jnp.int32)

            @plsc.parallel_loop(0, NBINS, unroll=8, carry=jnp.int32(0))
            def g_loop(d, g):
                g_s[d] = g
                return g + sums_s[d]

            @plsc.parallel_loop(0, NBINS, unroll=8)
            def addg_loop(d):
                ctr[d, :] = ctr[d, :] + jnp.full((L,), g_s[d], jnp.int32)

            def scat_body(v4, c):
                for u in range(4):
                    idx = lane_c + (v4 * 4 + u)
                    k = plsc.load_gather(kb_s, [idx])
                    val = plsc.load_gather(vb_s, [idx])
                    d = jnp.bitwise_and(lax.shift_right_logical(k, shift),
                                        jnp.int32(NBINS - 1))
                    pos = plsc.load_gather(ctr, [d, lane])
                    plsc.store_scatter(kb_d, [pos], k)
                    plsc.store_scatter(vb_d, [pos], val)
                    plsc.store_scatter(ctr, [d, lane], pos + 1)
                return c
            lax.fori_loop(0, chunk // 4, scat_body, 0)

        do_pass(key_a, val_a, key_b, val_b, 0)
        do_pass(key_b, val_b, key_a, val_a, 8)
        do_pass(key_a, val_a, key_b, val_b, 16)
        do_pass(key_b, val_b, key_a, val_a, 24)

        n_tail = (d_cnt + L - 1) // L

        @plsc.parallel_loop(0, n_tail, unroll=4)
        def tail_loop(j):
            t = jnp.flip(val_b[pl.ds(S + PAD - L * (j + 1), L)], axis=0)
            plsc.store_scatter(val_a, [m_cnt + j * L + lane], t)

        pltpu.sync_copy(val_a.at[pl.ds(0, S)], out_hbm.at[row])
        return carry_row

    lax.fori_loop(0, rows_per_w, do_row, 0)


@jax.jit
def kernel(query_tokens, rag_scores, attention_mask):
    info = plsc.get_sparse_core_info()
    num_workers = info.num_cores * info.num_subcores
    mesh = plsc.VectorSubcoreMesh(core_axis_name="c", subcore_axis_name="s")
    body = functools.partial(_body, num_workers=num_workers)
    fn = pl.kernel(
        body,
        out_type=jax.ShapeDtypeStruct((B, S), jnp.int32),
        mesh=mesh,
        compiler_params=pltpu.CompilerParams(needs_layout_passes=False),
        scratch_types=[
            pltpu.VMEM((S,), jnp.int32),
            pltpu.VMEM((S,), jnp.float32),
            pltpu.VMEM((S,), jnp.int32),
            pltpu.VMEM((S + PAD,), jnp.int32),
            pltpu.VMEM((S + PAD,), jnp.int32),
            pltpu.VMEM((S + PAD,), jnp.int32),
            pltpu.VMEM((S + PAD,), jnp.int32),
            pltpu.VMEM((NBINS, L), jnp.int32),
            pltpu.VMEM((NBINS, L), jnp.int32),
            pltpu.SMEM((NBINS,), jnp.int32),
            pltpu.SMEM((NBINS,), jnp.int32),
        ],
    )
    return fn(query_tokens, rag_scores, attention_mask)
